# baseline (speedup 1.0000x reference)
import functools

import jax
import jax.numpy as jnp
from jax import lax
from jax.experimental import pallas as pl
from jax.experimental.pallas import tpu as pltpu
from jax.experimental.pallas import tpu_sc as plsc

_B, _N, _D, _C, _K = 4, 8192, 768, 10, 256
_CP = 128
_CL = 16
_BN = 1024
_NROW, _NCOL = 64, 128
_NW = 32


def _proj_body(x_ref, w_ref, b_ref, lg_ref, key_ref):
    xb = x_ref[0]
    lgT = lax.dot_general(w_ref[...].astype(jnp.bfloat16),
                          xb.astype(jnp.bfloat16),
                          dimension_numbers=(((1,), (1,)), ((), ())),
                          preferred_element_type=jnp.float32)
    lgT = lgT[:_CL] + b_ref[:_CL]
    row = lax.broadcasted_iota(jnp.int32, (_CL, _BN), 0)
    valid = row < _C
    lmax = jnp.max(jnp.where(valid, lgT, -jnp.inf), axis=0, keepdims=True)
    e = jnp.where(valid, jnp.exp(lgT - lmax), 0.0)
    t = e[:8] + e[8:16]
    t = t[:4] + t[4:8]
    t = t[:2] + t[2:4]
    s = t[0:1] + t[1:2]
    p = e / s
    key_ref[0, 0] = jnp.max(p, axis=0, keepdims=True)
    lg_ref[0] = lgT


def _project(x, wp, bp):
    grid = (_B, _N // _BN)
    return pl.pallas_call(
        _proj_body,
        grid=grid,
        in_specs=[
            pl.BlockSpec((1, _BN, _D), lambda b, i: (b, i, 0)),
            pl.BlockSpec((_CP, _D), lambda b, i: (0, 0)),
            pl.BlockSpec((_CP, 1), lambda b, i: (0, 0)),
        ],
        out_specs=[
            pl.BlockSpec((1, _CL, _BN), lambda b, i: (b, 0, i)),
            pl.BlockSpec((1, 1, 1, _BN), lambda b, i: (b, i, 0, 0)),
        ],
        out_shape=[
            jax.ShapeDtypeStruct((_B, _CL, _N), jnp.float32),
            jax.ShapeDtypeStruct((_B, _N // _BN, 1, _BN), jnp.float32),
        ],
        compiler_params=pltpu.CompilerParams(
            dimension_semantics=("parallel", "arbitrary")),
    )(x, wp, bp)


def _sort_body(key_ref, ord_ref):
    key = key_ref[...]
    shape = (_B, _NROW, _NCOL)
    mi = lax.broadcasted_iota(jnp.int32, shape, 1)
    li = lax.broadcasted_iota(jnp.int32, shape, 2)
    flat = mi * _NCOL + li
    idx = flat

    def partner(v, j):
        if j >= _NCOL:
            ax, sh = 1, j // _NCOL
        else:
            ax, sh = 2, j
        lo = jnp.roll(v, -sh, axis=ax)
        hi = jnp.roll(v, sh, axis=ax)
        return lo, hi

    kk = 2
    while kk <= _N:
        j = kk // 2
        while j >= 1:
            lmask = (flat & j) == 0
            desc = (flat & kk) == 0
            klo, khi = partner(key, j)
            ilo, ihi = partner(idx, j)
            pk = jnp.where(lmask, klo, khi)
            pi = jnp.where(lmask, ilo, ihi)
            precedes = (key > pk) | ((key == pk) & (idx < pi))
            keep = precedes ^ desc ^ lmask
            key = jnp.where(keep, key, pk)
            idx = jnp.where(keep, idx, pi)
            j //= 2
        kk *= 2

    bi = lax.broadcasted_iota(jnp.int32, shape, 0)
    ord_ref[...] = idx + bi * _N


def _sort(key):
    return pl.pallas_call(
        _sort_body,
        out_shape=jax.ShapeDtypeStruct((_B, _NROW, _NCOL), jnp.int32),
    )(key)


_SEL_PW = (_B * _K) // _NW
_PRD_PW = (_B * _N) // _NW


def _gath_body(x_hbm, lg_hbm, selid_hbm, ord_hbm, sel_out, prd_out,
               idxa_v, rowsa_v, sema, idxb_v, rowsb_v, semb):
    wid = lax.axis_index("s") * 2 + lax.axis_index("c")
    basea = wid * _SEL_PW
    baseb = wid * _PRD_PW
    pltpu.sync_copy(selid_hbm.at[pl.ds(basea, _SEL_PW)], idxa_v)
    pltpu.sync_copy(ord_hbm.at[pl.ds(baseb, _PRD_PW)], idxb_v)
    ca = pltpu.async_copy(x_hbm.at[idxa_v], rowsa_v, sema)
    cb = pltpu.async_copy(lg_hbm.at[idxb_v], rowsb_v, semb)
    ca.wait()
    pltpu.sync_copy(rowsa_v, sel_out.at[pl.ds(basea, _SEL_PW)])
    cb.wait()
    pltpu.sync_copy(rowsb_v, prd_out.at[pl.ds(baseb, _PRD_PW)])


def _gather_both(x_flat, lg_flat, selid, ord_flat):
    mesh = plsc.VectorSubcoreMesh(
        core_axis_name="c", subcore_axis_name="s", num_cores=2)
    fn = pl.kernel(
        _gath_body,
        mesh=mesh,
        out_type=[
            jax.ShapeDtypeStruct((_B * _K, _D), jnp.float32),
            jax.ShapeDtypeStruct((_B * _N, _CL), jnp.float32),
        ],
        scratch_types=[
            pltpu.VMEM((_SEL_PW,), jnp.int32),
            pltpu.VMEM((_SEL_PW, _D), jnp.float32),
            pltpu.SemaphoreType.DMA,
            pltpu.VMEM((_PRD_PW,), jnp.int32),
            pltpu.VMEM((_PRD_PW, _CL), jnp.float32),
            pltpu.SemaphoreType.DMA,
        ],
        compiler_params=pltpu.CompilerParams(use_tc_tiling_on_sc=False),
    )
    return fn(x_flat, lg_flat, selid, ord_flat)


def kernel(x, W, b):
    wp = jnp.zeros((_CP, _D), jnp.float32).at[:_C].set(W)
    bp = jnp.zeros((_CP, 1), jnp.float32).at[:_C].set(b[:, None])
    lgT, key4 = _project(x, wp, bp)
    lg16 = lgT.transpose(0, 2, 1)
    key = key4.reshape(_B, _NROW, _NCOL)
    ordg = _sort(key).reshape(_B, _N)
    selid = ordg[:, :_K].reshape(_B * _K)
    ord_flat = ordg.reshape(_B * _N)
    sel, prd = _gather_both(x.reshape(_B * _N, _D),
                            lg16.reshape(_B * _N, _CL), selid, ord_flat)
    selections = sel.reshape(_B, _K, _D)
    preds = prd.reshape(_B, _N, _CL)[:, :, :_C]
    return selections, preds[:, :_K], preds[:, _K:]


# speedup vs baseline: 1.7014x; 1.7014x over previous
import functools

import jax
import jax.numpy as jnp
from jax import lax
from jax.experimental import pallas as pl
from jax.experimental.pallas import tpu as pltpu
from jax.experimental.pallas import tpu_sc as plsc

_B, _N, _D, _C, _K = 4, 8192, 768, 10, 256
_CP = 128
_CL = 16
_BN = 1024
_NROW, _NCOL = 64, 128
_NW = 32


def _proj_body(x_ref, w_ref, b_ref, lg_ref, key_ref):
    xb = x_ref[0]
    lgT = lax.dot_general(w_ref[...].astype(jnp.bfloat16),
                          xb.astype(jnp.bfloat16),
                          dimension_numbers=(((1,), (1,)), ((), ())),
                          preferred_element_type=jnp.float32)
    lgT = lgT[:_CL] + b_ref[:_CL]
    row = lax.broadcasted_iota(jnp.int32, (_CL, _BN), 0)
    valid = row < _C
    lmax = jnp.max(jnp.where(valid, lgT, -jnp.inf), axis=0, keepdims=True)
    e = jnp.where(valid, jnp.exp(lgT - lmax), 0.0)
    t = e[:8] + e[8:16]
    t = t[:4] + t[4:8]
    t = t[:2] + t[2:4]
    s = t[0:1] + t[1:2]
    p = e / s
    key_ref[0, 0] = jnp.max(p, axis=0, keepdims=True)
    lg_ref[0] = lgT


def _project(x, wp, bp):
    grid = (_B, _N // _BN)
    return pl.pallas_call(
        _proj_body,
        grid=grid,
        in_specs=[
            pl.BlockSpec((1, _BN, _D), lambda b, i: (b, i, 0)),
            pl.BlockSpec((_CP, _D), lambda b, i: (0, 0)),
            pl.BlockSpec((_CP, 1), lambda b, i: (0, 0)),
        ],
        out_specs=[
            pl.BlockSpec((1, _CL, _BN), lambda b, i: (b, 0, i)),
            pl.BlockSpec((1, 1, 1, _BN), lambda b, i: (b, i, 0, 0)),
        ],
        out_shape=[
            jax.ShapeDtypeStruct((_B, _CL, _N), jnp.float32),
            jax.ShapeDtypeStruct((_B, _N // _BN, 1, _BN), jnp.float32),
        ],
        compiler_params=pltpu.CompilerParams(
            dimension_semantics=("parallel", "arbitrary")),
    )(x, wp, bp)


def _sort_body(key_ref, ord_ref):
    key = key_ref[...]
    shape = (_B, _NROW, _NCOL)
    mi = lax.broadcasted_iota(jnp.int32, shape, 1)
    li = lax.broadcasted_iota(jnp.int32, shape, 2)
    flat = mi * _NCOL + li
    idx = flat

    def partner(v, j):
        if j >= _NCOL:
            ax, sh = 1, j // _NCOL
        else:
            ax, sh = 2, j
        lo = jnp.roll(v, -sh, axis=ax)
        hi = jnp.roll(v, sh, axis=ax)
        return lo, hi

    kk = 2
    while kk <= _N:
        j = kk // 2
        while j >= 1:
            lmask = (flat & j) == 0
            desc = (flat & kk) == 0
            klo, khi = partner(key, j)
            ilo, ihi = partner(idx, j)
            pk = jnp.where(lmask, klo, khi)
            pi = jnp.where(lmask, ilo, ihi)
            precedes = (key > pk) | ((key == pk) & (idx < pi))
            keep = precedes ^ desc ^ lmask
            key = jnp.where(keep, key, pk)
            idx = jnp.where(keep, idx, pi)
            j //= 2
        kk *= 2

    bi = lax.broadcasted_iota(jnp.int32, shape, 0)
    ord_ref[...] = idx + bi * _N


def _sort(key):
    return pl.pallas_call(
        _sort_body,
        out_shape=jax.ShapeDtypeStruct((_B, _NROW, _NCOL), jnp.int32),
    )(key)


_SEL_PW = (_B * _K) // _NW
_PRD_PW = (_B * _N) // _NW


def _sel_body(x_hbm, selid_hbm, sel_out, idxa_v, rowsa_v, sema):
    wid = lax.axis_index("s") * 2 + lax.axis_index("c")
    basea = wid * _SEL_PW
    pltpu.sync_copy(selid_hbm.at[pl.ds(basea, _SEL_PW)], idxa_v)
    pltpu.async_copy(x_hbm.at[idxa_v], rowsa_v, sema).wait()
    pltpu.sync_copy(rowsa_v, sel_out.at[pl.ds(basea, _SEL_PW)])


def _gather_sel(x_flat, selid):
    mesh = plsc.VectorSubcoreMesh(
        core_axis_name="c", subcore_axis_name="s", num_cores=2)
    fn = pl.kernel(
        _sel_body,
        mesh=mesh,
        out_type=jax.ShapeDtypeStruct((_B * _K, _D), jnp.float32),
        scratch_types=[
            pltpu.VMEM((_SEL_PW,), jnp.int32),
            pltpu.VMEM((_SEL_PW, _D), jnp.float32),
            pltpu.SemaphoreType.DMA,
        ],
    )
    return fn(x_flat, selid)


_NSTREAM = 4
_PRD_CH = _PRD_PW // _NSTREAM


def _prd_body(lg_hbm, ord_hbm, prd_out, idxb_v, rowsb_v, *sems):
    wid = lax.axis_index("s") * 2 + lax.axis_index("c")
    baseb = wid * _PRD_PW
    pltpu.sync_copy(ord_hbm.at[pl.ds(baseb, _PRD_PW)], idxb_v)
    copies = []
    for c in range(_NSTREAM):
        copies.append(pltpu.async_copy(
            lg_hbm.at[idxb_v.at[pl.ds(c * _PRD_CH, _PRD_CH)]],
            rowsb_v.at[pl.ds(c * _PRD_CH, _PRD_CH)], sems[c]))
    for c in range(_NSTREAM):
        copies[c].wait()
    pltpu.sync_copy(rowsb_v, prd_out.at[pl.ds(baseb, _PRD_PW)])


def _gather_prd(lg_flat, ord_flat):
    mesh = plsc.VectorSubcoreMesh(
        core_axis_name="c", subcore_axis_name="s", num_cores=2)
    fn = pl.kernel(
        _prd_body,
        mesh=mesh,
        out_type=jax.ShapeDtypeStruct((_B * _N, _CL), jnp.float32),
        scratch_types=[
            pltpu.VMEM((_PRD_PW,), jnp.int32),
            pltpu.VMEM((_PRD_PW, _CL), jnp.float32),
        ] + [pltpu.SemaphoreType.DMA] * _NSTREAM,
        compiler_params=pltpu.CompilerParams(use_tc_tiling_on_sc=False),
    )
    return fn(lg_flat, ord_flat)


def kernel(x, W, b):
    wp = jnp.zeros((_CP, _D), jnp.float32).at[:_C].set(W)
    bp = jnp.zeros((_CP, 1), jnp.float32).at[:_C].set(b[:, None])
    lgT, key4 = _project(x, wp, bp)
    lg16 = lgT.transpose(0, 2, 1)
    key = key4.reshape(_B, _NROW, _NCOL)
    ordg = _sort(key).reshape(_B, _N)
    selid = ordg[:, :_K].reshape(_B * _K)
    ord_flat = ordg.reshape(_B * _N)
    sel = _gather_sel(x.reshape(_B * _N, _D), selid)
    prd = _gather_prd(lg16.reshape(_B * _N, _CL), ord_flat)
    selections = sel.reshape(_B, _K, _D)
    preds = prd.reshape(_B, _N, _CL)[:, :, :_C]
    return selections, preds[:, :_K], preds[:, _K:]


# speedup vs baseline: 1.7650x; 1.0374x over previous
import functools

import jax
import jax.numpy as jnp
from jax import lax
from jax.experimental import pallas as pl
from jax.experimental.pallas import tpu as pltpu
from jax.experimental.pallas import tpu_sc as plsc

_B, _N, _D, _C, _K = 4, 8192, 768, 10, 256
_CP = 128
_CL = 16
_BN = 1024
_NROW, _NCOL = 64, 128
_NW = 32


def _proj_body(x_ref, w_ref, b_ref, lg_ref, key_ref):
    xb = x_ref[0]
    lgT = lax.dot_general(w_ref[...].astype(jnp.bfloat16),
                          xb.astype(jnp.bfloat16),
                          dimension_numbers=(((1,), (1,)), ((), ())),
                          preferred_element_type=jnp.float32)
    lgT = lgT[:_CL] + b_ref[:_CL]
    row = lax.broadcasted_iota(jnp.int32, (_CL, _BN), 0)
    valid = row < _C
    lmax = jnp.max(jnp.where(valid, lgT, -jnp.inf), axis=0, keepdims=True)
    e = jnp.where(valid, jnp.exp(lgT - lmax), 0.0)
    t = e[:8] + e[8:16]
    t = t[:4] + t[4:8]
    t = t[:2] + t[2:4]
    s = t[0:1] + t[1:2]
    p = e / s
    key_ref[0, 0] = jnp.max(p, axis=0, keepdims=True)
    lg_ref[0] = lgT.T


def _project(x, wp, bp):
    grid = (_B, _N // _BN)
    return pl.pallas_call(
        _proj_body,
        grid=grid,
        in_specs=[
            pl.BlockSpec((1, _BN, _D), lambda b, i: (b, i, 0)),
            pl.BlockSpec((_CP, _D), lambda b, i: (0, 0)),
            pl.BlockSpec((_CP, 1), lambda b, i: (0, 0)),
        ],
        out_specs=[
            pl.BlockSpec((1, _BN, _CL), lambda b, i: (b, i, 0)),
            pl.BlockSpec((1, 1, 1, _BN), lambda b, i: (b, i, 0, 0)),
        ],
        out_shape=[
            jax.ShapeDtypeStruct((_B, _N, _CL), jnp.float32),
            jax.ShapeDtypeStruct((_B, _N // _BN, 1, _BN), jnp.float32),
        ],
        compiler_params=pltpu.CompilerParams(
            dimension_semantics=("parallel", "arbitrary")),
    )(x, wp, bp)


def _sort_body(key_ref, ord_ref):
    key = key_ref[...]
    shape = (_B, _NROW, _NCOL)
    mi = lax.broadcasted_iota(jnp.int32, shape, 1)
    li = lax.broadcasted_iota(jnp.int32, shape, 2)
    flat = mi * _NCOL + li
    idx = flat

    def partner(v, j):
        if j >= _NCOL:
            ax, sh = 1, j // _NCOL
        else:
            ax, sh = 2, j
        lo = jnp.roll(v, -sh, axis=ax)
        hi = jnp.roll(v, sh, axis=ax)
        return lo, hi

    kk = 2
    while kk <= _N:
        j = kk // 2
        while j >= 1:
            lmask = (flat & j) == 0
            desc = (flat & kk) == 0
            klo, khi = partner(key, j)
            ilo, ihi = partner(idx, j)
            pk = jnp.where(lmask, klo, khi)
            pi = jnp.where(lmask, ilo, ihi)
            precedes = (key > pk) | ((key == pk) & (idx < pi))
            keep = precedes ^ desc ^ lmask
            key = jnp.where(keep, key, pk)
            idx = jnp.where(keep, idx, pi)
            j //= 2
        kk *= 2

    bi = lax.broadcasted_iota(jnp.int32, shape, 0)
    ord_ref[...] = idx + bi * _N


def _sort(key):
    return pl.pallas_call(
        _sort_body,
        out_shape=jax.ShapeDtypeStruct((_B, _NROW, _NCOL), jnp.int32),
    )(key)


_SEL_PW = (_B * _K) // _NW
_PRD_PW = (_B * _N) // _NW


def _sel_body(x_hbm, selid_hbm, sel_out, idxa_v, rowsa_v, sema):
    wid = lax.axis_index("s") * 2 + lax.axis_index("c")
    basea = wid * _SEL_PW
    pltpu.sync_copy(selid_hbm.at[pl.ds(basea, _SEL_PW)], idxa_v)
    pltpu.async_copy(x_hbm.at[idxa_v], rowsa_v, sema).wait()
    pltpu.sync_copy(rowsa_v, sel_out.at[pl.ds(basea, _SEL_PW)])


def _gather_sel(x_flat, selid):
    mesh = plsc.VectorSubcoreMesh(
        core_axis_name="c", subcore_axis_name="s", num_cores=2)
    fn = pl.kernel(
        _sel_body,
        mesh=mesh,
        out_type=jax.ShapeDtypeStruct((_B * _K, _D), jnp.float32),
        scratch_types=[
            pltpu.VMEM((_SEL_PW,), jnp.int32),
            pltpu.VMEM((_SEL_PW, _D), jnp.float32),
            pltpu.SemaphoreType.DMA,
        ],
    )
    return fn(x_flat, selid)


_NSTREAM = 4
_PRD_CH = _PRD_PW // _NSTREAM


def _prd_body(lg_hbm, ord_hbm, prd_out, idxb_v, rowsb_v, *sems):
    wid = lax.axis_index("s") * 2 + lax.axis_index("c")
    baseb = wid * _PRD_PW
    pltpu.sync_copy(ord_hbm.at[pl.ds(baseb, _PRD_PW)], idxb_v)
    copies = []
    for c in range(_NSTREAM):
        copies.append(pltpu.async_copy(
            lg_hbm.at[idxb_v.at[pl.ds(c * _PRD_CH, _PRD_CH)]],
            rowsb_v.at[pl.ds(c * _PRD_CH, _PRD_CH)], sems[c]))
    for c in range(_NSTREAM):
        copies[c].wait()
    pltpu.sync_copy(rowsb_v, prd_out.at[pl.ds(baseb, _PRD_PW)])


def _gather_prd(lg_flat, ord_flat):
    mesh = plsc.VectorSubcoreMesh(
        core_axis_name="c", subcore_axis_name="s", num_cores=2)
    fn = pl.kernel(
        _prd_body,
        mesh=mesh,
        out_type=jax.ShapeDtypeStruct((_B * _N, _CL), jnp.float32),
        scratch_types=[
            pltpu.VMEM((_PRD_PW,), jnp.int32),
            pltpu.VMEM((_PRD_PW, _CL), jnp.float32),
        ] + [pltpu.SemaphoreType.DMA] * _NSTREAM,
        compiler_params=pltpu.CompilerParams(use_tc_tiling_on_sc=False),
    )
    return fn(lg_flat, ord_flat)


def kernel(x, W, b):
    wp = jnp.zeros((_CP, _D), jnp.float32).at[:_C].set(W)
    bp = jnp.zeros((_CP, 1), jnp.float32).at[:_C].set(b[:, None])
    lg16, key4 = _project(x, wp, bp)
    key = key4.reshape(_B, _NROW, _NCOL)
    ordg = _sort(key).reshape(_B, _N)
    selid = ordg[:, :_K].reshape(_B * _K)
    ord_flat = ordg.reshape(_B * _N)
    sel = _gather_sel(x.reshape(_B * _N, _D), selid)
    prd = _gather_prd(lg16.reshape(_B * _N, _CL), ord_flat)
    selections = sel.reshape(_B, _K, _D)
    preds = prd.reshape(_B, _N, _CL)[:, :, :_C]
    return selections, preds[:, :_K], preds[:, _K:]


# speedup vs baseline: 1.8948x; 1.0735x over previous
import functools

import jax
import jax.numpy as jnp
from jax import lax
from jax.experimental import pallas as pl
from jax.experimental.pallas import tpu as pltpu
from jax.experimental.pallas import tpu_sc as plsc

_B, _N, _D, _C, _K = 4, 8192, 768, 10, 256
_CP = 128
_CL = 16
_BN = 2048
_NROW, _NCOL = 64, 128
_NW = 32


def _proj_body(x_ref, w_ref, b_ref, lg_ref, key_ref):
    xb = x_ref[0]
    lgT = lax.dot_general(w_ref[...].astype(jnp.bfloat16),
                          xb.astype(jnp.bfloat16),
                          dimension_numbers=(((1,), (1,)), ((), ())),
                          preferred_element_type=jnp.float32)
    lgT = lgT[:_CL] + b_ref[:_CL]
    row = lax.broadcasted_iota(jnp.int32, (_CL, _BN), 0)
    valid = row < _C
    lmax = jnp.max(jnp.where(valid, lgT, -jnp.inf), axis=0, keepdims=True)
    e = jnp.where(valid, jnp.exp(lgT - lmax), 0.0)
    t = e[:8] + e[8:16]
    t = t[:4] + t[4:8]
    t = t[:2] + t[2:4]
    s = t[0:1] + t[1:2]
    p = e / s
    key_ref[0, 0] = jnp.max(p, axis=0, keepdims=True)
    lg_ref[0] = lgT.T


def _project(x, wp, bp):
    grid = (_B, _N // _BN)
    return pl.pallas_call(
        _proj_body,
        grid=grid,
        in_specs=[
            pl.BlockSpec((1, _BN, _D), lambda b, i: (b, i, 0)),
            pl.BlockSpec((_CP, _D), lambda b, i: (0, 0)),
            pl.BlockSpec((_CP, 1), lambda b, i: (0, 0)),
        ],
        out_specs=[
            pl.BlockSpec((1, _BN, _CL), lambda b, i: (b, i, 0)),
            pl.BlockSpec((1, 1, 1, _BN), lambda b, i: (b, i, 0, 0)),
        ],
        out_shape=[
            jax.ShapeDtypeStruct((_B, _N, _CL), jnp.float32),
            jax.ShapeDtypeStruct((_B, _N // _BN, 1, _BN), jnp.float32),
        ],
        compiler_params=pltpu.CompilerParams(
            dimension_semantics=("parallel", "arbitrary")),
    )(x, wp, bp)


def _sort_body(key_ref, ord_ref):
    key = key_ref[...]
    shape = (_B, _NROW, _NCOL)
    mi = lax.broadcasted_iota(jnp.int32, shape, 1)
    li = lax.broadcasted_iota(jnp.int32, shape, 2)
    flat = mi * _NCOL + li
    idx = flat

    def partner(v, j):
        if j >= _NCOL:
            ax, sh = 1, j // _NCOL
        else:
            ax, sh = 2, j
        lo = jnp.roll(v, -sh, axis=ax)
        hi = jnp.roll(v, sh, axis=ax)
        return lo, hi

    kk = 2
    while kk <= _N:
        j = kk // 2
        while j >= 1:
            lmask = (flat & j) == 0
            desc = (flat & kk) == 0
            klo, khi = partner(key, j)
            ilo, ihi = partner(idx, j)
            pk = jnp.where(lmask, klo, khi)
            pi = jnp.where(lmask, ilo, ihi)
            precedes = (key > pk) | ((key == pk) & (idx < pi))
            keep = precedes ^ desc ^ lmask
            key = jnp.where(keep, key, pk)
            idx = jnp.where(keep, idx, pi)
            j //= 2
        kk *= 2

    bi = lax.broadcasted_iota(jnp.int32, shape, 0)
    ord_ref[...] = idx + bi * _N


def _sort(key):
    return pl.pallas_call(
        _sort_body,
        out_shape=jax.ShapeDtypeStruct((_B, _NROW, _NCOL), jnp.int32),
    )(key)


_SEL_PW = (_B * _K) // _NW
_PRD_PW = (_B * _N) // _NW


def _sel_body(x_hbm, selid_hbm, sel_out, idxa_v, rowsa_v, sema):
    wid = lax.axis_index("s") * 2 + lax.axis_index("c")
    basea = wid * _SEL_PW
    pltpu.sync_copy(selid_hbm.at[pl.ds(basea, _SEL_PW)], idxa_v)
    pltpu.async_copy(x_hbm.at[idxa_v], rowsa_v, sema).wait()
    pltpu.sync_copy(rowsa_v, sel_out.at[pl.ds(basea, _SEL_PW)])


def _gather_sel(x_flat, selid):
    mesh = plsc.VectorSubcoreMesh(
        core_axis_name="c", subcore_axis_name="s", num_cores=2)
    fn = pl.kernel(
        _sel_body,
        mesh=mesh,
        out_type=jax.ShapeDtypeStruct((_B * _K, _D), jnp.float32),
        scratch_types=[
            pltpu.VMEM((_SEL_PW,), jnp.int32),
            pltpu.VMEM((_SEL_PW, _D), jnp.float32),
            pltpu.SemaphoreType.DMA,
        ],
    )
    return fn(x_flat, selid)


_NSTREAM = 4
_PRD_CH = _PRD_PW // _NSTREAM


def _prd_body(lg_hbm, ord_hbm, prd_out, idxb_v, rowsb_v, *sems):
    wid = lax.axis_index("s") * 2 + lax.axis_index("c")
    baseb = wid * _PRD_PW
    pltpu.sync_copy(ord_hbm.at[pl.ds(baseb, _PRD_PW)], idxb_v)
    copies = []
    for c in range(_NSTREAM):
        copies.append(pltpu.async_copy(
            lg_hbm.at[idxb_v.at[pl.ds(c * _PRD_CH, _PRD_CH)]],
            rowsb_v.at[pl.ds(c * _PRD_CH, _PRD_CH)], sems[c]))
    for c in range(_NSTREAM):
        copies[c].wait()
    pltpu.sync_copy(rowsb_v, prd_out.at[pl.ds(baseb, _PRD_PW)])


def _gather_prd(lg_flat, ord_flat):
    mesh = plsc.VectorSubcoreMesh(
        core_axis_name="c", subcore_axis_name="s", num_cores=2)
    fn = pl.kernel(
        _prd_body,
        mesh=mesh,
        out_type=jax.ShapeDtypeStruct((_B * _N, _CL), jnp.float32),
        scratch_types=[
            pltpu.VMEM((_PRD_PW,), jnp.int32),
            pltpu.VMEM((_PRD_PW, _CL), jnp.float32),
        ] + [pltpu.SemaphoreType.DMA] * _NSTREAM,
        compiler_params=pltpu.CompilerParams(use_tc_tiling_on_sc=False),
    )
    return fn(lg_flat, ord_flat)


def kernel(x, W, b):
    wp = jnp.zeros((_CP, _D), jnp.float32).at[:_C].set(W)
    bp = jnp.zeros((_CP, 1), jnp.float32).at[:_C].set(b[:, None])
    lg16, key4 = _project(x, wp, bp)
    key = key4.reshape(_B, _NROW, _NCOL)
    ordg = _sort(key).reshape(_B, _N)
    selid = ordg[:, :_K].reshape(_B * _K)
    ord_flat = ordg.reshape(_B * _N)
    sel = _gather_sel(x.reshape(_B * _N, _D), selid)
    prd = _gather_prd(lg16.reshape(_B * _N, _CL), ord_flat)
    selections = sel.reshape(_B, _K, _D)
    preds = prd.reshape(_B, _N, _CL)[:, :, :_C]
    return selections, preds[:, :_K], preds[:, _K:]


# speedup vs baseline: 1.9262x; 1.0166x over previous
import functools

import jax
import jax.numpy as jnp
from jax import lax
from jax.experimental import pallas as pl
from jax.experimental.pallas import tpu as pltpu
from jax.experimental.pallas import tpu_sc as plsc

_B, _N, _D, _C, _K = 4, 8192, 768, 10, 256
_CP = 128
_CL = 16
_BN = 4096
_NROW, _NCOL = 64, 128
_NW = 32


def _proj_body(x_ref, w_ref, b_ref, lg_ref, key_ref):
    xb = x_ref[0]
    lgT = lax.dot_general(w_ref[...].astype(jnp.bfloat16),
                          xb.astype(jnp.bfloat16),
                          dimension_numbers=(((1,), (1,)), ((), ())),
                          preferred_element_type=jnp.float32)
    lgT = lgT[:_CL] + b_ref[:_CL]
    row = lax.broadcasted_iota(jnp.int32, (_CL, _BN), 0)
    valid = row < _C
    lmax = jnp.max(jnp.where(valid, lgT, -jnp.inf), axis=0, keepdims=True)
    e = jnp.where(valid, jnp.exp(lgT - lmax), 0.0)
    t = e[:8] + e[8:16]
    t = t[:4] + t[4:8]
    t = t[:2] + t[2:4]
    s = t[0:1] + t[1:2]
    p = e / s
    key_ref[0, 0] = jnp.max(p, axis=0, keepdims=True)
    lg_ref[0] = lgT.T


def _project(x, wp, bp):
    grid = (_B, _N // _BN)
    return pl.pallas_call(
        _proj_body,
        grid=grid,
        in_specs=[
            pl.BlockSpec((1, _BN, _D), lambda b, i: (b, i, 0)),
            pl.BlockSpec((_CP, _D), lambda b, i: (0, 0)),
            pl.BlockSpec((_CP, 1), lambda b, i: (0, 0)),
        ],
        out_specs=[
            pl.BlockSpec((1, _BN, _CL), lambda b, i: (b, i, 0)),
            pl.BlockSpec((1, 1, 1, _BN), lambda b, i: (b, i, 0, 0)),
        ],
        out_shape=[
            jax.ShapeDtypeStruct((_B, _N, _CL), jnp.float32),
            jax.ShapeDtypeStruct((_B, _N // _BN, 1, _BN), jnp.float32),
        ],
        compiler_params=pltpu.CompilerParams(
            dimension_semantics=("parallel", "arbitrary")),
    )(x, wp, bp)


def _sort_body(key_ref, ord_ref):
    key = key_ref[...]
    shape = (_B, _NROW, _NCOL)
    mi = lax.broadcasted_iota(jnp.int32, shape, 1)
    li = lax.broadcasted_iota(jnp.int32, shape, 2)
    flat = mi * _NCOL + li
    idx = flat

    def partner(v, j):
        if j >= _NCOL:
            ax, sh = 1, j // _NCOL
        else:
            ax, sh = 2, j
        lo = jnp.roll(v, -sh, axis=ax)
        hi = jnp.roll(v, sh, axis=ax)
        return lo, hi

    kk = 2
    while kk <= _N:
        j = kk // 2
        while j >= 1:
            lmask = (flat & j) == 0
            desc = (flat & kk) == 0
            klo, khi = partner(key, j)
            ilo, ihi = partner(idx, j)
            pk = jnp.where(lmask, klo, khi)
            pi = jnp.where(lmask, ilo, ihi)
            precedes = (key > pk) | ((key == pk) & (idx < pi))
            keep = precedes ^ desc ^ lmask
            key = jnp.where(keep, key, pk)
            idx = jnp.where(keep, idx, pi)
            j //= 2
        kk *= 2

    bi = lax.broadcasted_iota(jnp.int32, shape, 0)
    ord_ref[...] = idx + bi * _N


def _sort(key):
    return pl.pallas_call(
        _sort_body,
        out_shape=jax.ShapeDtypeStruct((_B, _NROW, _NCOL), jnp.int32),
    )(key)


_SEL_PW = (_B * _K) // _NW
_PRD_PW = (_B * _N) // _NW


def _sel_body(x_hbm, selid_hbm, sel_out, idxa_v, rowsa_v, sema):
    wid = lax.axis_index("s") * 2 + lax.axis_index("c")
    basea = wid * _SEL_PW
    pltpu.sync_copy(selid_hbm.at[pl.ds(basea, _SEL_PW)], idxa_v)
    pltpu.async_copy(x_hbm.at[idxa_v], rowsa_v, sema).wait()
    pltpu.sync_copy(rowsa_v, sel_out.at[pl.ds(basea, _SEL_PW)])


def _gather_sel(x_flat, selid):
    mesh = plsc.VectorSubcoreMesh(
        core_axis_name="c", subcore_axis_name="s", num_cores=2)
    fn = pl.kernel(
        _sel_body,
        mesh=mesh,
        out_type=jax.ShapeDtypeStruct((_B * _K, _D), jnp.float32),
        scratch_types=[
            pltpu.VMEM((_SEL_PW,), jnp.int32),
            pltpu.VMEM((_SEL_PW, _D), jnp.float32),
            pltpu.SemaphoreType.DMA,
        ],
    )
    return fn(x_flat, selid)


_NSTREAM = 4
_PRD_CH = _PRD_PW // _NSTREAM


def _prd_body(lg_hbm, ord_hbm, prd_out, idxb_v, rowsb_v, *sems):
    wid = lax.axis_index("s") * 2 + lax.axis_index("c")
    baseb = wid * _PRD_PW
    pltpu.sync_copy(ord_hbm.at[pl.ds(baseb, _PRD_PW)], idxb_v)
    copies = []
    for c in range(_NSTREAM):
        copies.append(pltpu.async_copy(
            lg_hbm.at[idxb_v.at[pl.ds(c * _PRD_CH, _PRD_CH)]],
            rowsb_v.at[pl.ds(c * _PRD_CH, _PRD_CH)], sems[c]))
    for c in range(_NSTREAM):
        copies[c].wait()
    pltpu.sync_copy(rowsb_v, prd_out.at[pl.ds(baseb, _PRD_PW)])


def _gather_prd(lg_flat, ord_flat):
    mesh = plsc.VectorSubcoreMesh(
        core_axis_name="c", subcore_axis_name="s", num_cores=2)
    fn = pl.kernel(
        _prd_body,
        mesh=mesh,
        out_type=jax.ShapeDtypeStruct((_B * _N, _CL), jnp.float32),
        scratch_types=[
            pltpu.VMEM((_PRD_PW,), jnp.int32),
            pltpu.VMEM((_PRD_PW, _CL), jnp.float32),
        ] + [pltpu.SemaphoreType.DMA] * _NSTREAM,
        compiler_params=pltpu.CompilerParams(use_tc_tiling_on_sc=False),
    )
    return fn(lg_flat, ord_flat)


def kernel(x, W, b):
    wp = jnp.zeros((_CP, _D), jnp.float32).at[:_C].set(W)
    bp = jnp.zeros((_CP, 1), jnp.float32).at[:_C].set(b[:, None])
    lg16, key4 = _project(x, wp, bp)
    key = key4.reshape(_B, _NROW, _NCOL)
    ordg = _sort(key).reshape(_B, _N)
    selid = ordg[:, :_K].reshape(_B * _K)
    ord_flat = ordg.reshape(_B * _N)
    sel = _gather_sel(x.reshape(_B * _N, _D), selid)
    prd = _gather_prd(lg16.reshape(_B * _N, _CL), ord_flat)
    selections = sel.reshape(_B, _K, _D)
    preds = prd.reshape(_B, _N, _CL)[:, :, :_C]
    return selections, preds[:, :_K], preds[:, _K:]
